# Initial kernel scaffold; baseline (speedup 1.0000x reference)
import jax
import jax.numpy as jnp
from jax.experimental import pallas as pl


def kernel(x, edge_index, W1, b1, W2, b2):
    raise NotImplementedError("write your pallas kernel here")


# speedup vs baseline: 9.9637x; 9.9637x over previous
import functools

import jax
import jax.numpy as jnp
from jax import lax
from jax.experimental import pallas as pl
from jax.experimental.pallas import tpu as pltpu
from jax.experimental.pallas import tpu_sc as plsc

NC = 2
NS = 16
LANE = 128
BI = 8


def _mesh():
    return plsc.VectorSubcoreMesh(
        core_axis_name="c", subcore_axis_name="s", num_cores=NC, num_subcores=NS
    )


def _make_deg_kernel(n_pad, erows):
    erows_w = erows // (NC * NS)
    rows_t = n_pad // NS

    @functools.partial(
        pl.kernel,
        out_type=jax.ShapeDtypeStruct((NC, n_pad), jnp.float32),
        mesh=_mesh(),
        scratch_types=[
            pltpu.VMEM((erows_w, LANE), jnp.int32),
            pltpu.VMEM((LANE,), jnp.float32),
            pltpu.VMEM_SHARED((n_pad,), jnp.float32),
        ],
    )
    def kdeg(dst_hbm, halves_hbm, ones_hbm, out_hbm, didx, onesv, acc):
        cid = lax.axis_index("c")
        sid = lax.axis_index("s")
        w = cid * NS + sid
        pltpu.sync_copy(dst_hbm.at[pl.ds(w * erows_w, erows_w)], didx)
        pltpu.sync_copy(ones_hbm, onesv)
        pltpu.sync_copy(
            halves_hbm.at[pl.ds(sid * rows_t, rows_t)],
            acc.at[pl.ds(sid * rows_t, rows_t)],
        )
        plsc.subcore_barrier()

        @pl.loop(0, erows_w)
        def _(i):
            pltpu.sync_copy(onesv, acc.at[didx.at[i]], add=True)

        plsc.subcore_barrier()
        pltpu.sync_copy(
            acc.at[pl.ds(sid * rows_t, rows_t)],
            out_hbm.at[cid, pl.ds(sid * rows_t, rows_t)],
        )

    return kdeg


def _make_edge_scatter(n_pad, erows, feat):
    erows_w = erows // (NC * NS)
    rows_t = n_pad // NS
    nblk = erows_w // BI

    @functools.partial(
        pl.kernel,
        out_type=jax.ShapeDtypeStruct((NC, n_pad, feat), jnp.float32),
        mesh=_mesh(),
        scratch_types=[
            pltpu.VMEM((erows_w, LANE), jnp.int32),
            pltpu.VMEM((BI, LANE), jnp.int32),
            pltpu.VMEM((BI, LANE), jnp.int32),
            pltpu.VMEM((LANE, feat), jnp.float32),
            pltpu.VMEM((LANE, feat), jnp.float32),
            pltpu.VMEM_SHARED((n_pad, feat), jnp.float32),
            pltpu.SemaphoreType.DMA,
            pltpu.SemaphoreType.DMA,
            pltpu.SemaphoreType.DMA,
        ],
    )
    def kscat(src_hbm, dst_hbm, z_hbm, zeros_hbm, out_hbm,
              sidx, dblk_a, dblk_b, gb0, gb1, acc, sem0, sem1, semd):
        cid = lax.axis_index("c")
        sid = lax.axis_index("s")
        w = cid * NS + sid
        base = w * erows_w
        pltpu.sync_copy(src_hbm.at[pl.ds(base, erows_w)], sidx)
        pltpu.sync_copy(dst_hbm.at[pl.ds(base, BI)], dblk_a)
        pltpu.sync_copy(
            zeros_hbm.at[pl.ds(sid * rows_t, rows_t)],
            acc.at[pl.ds(sid * rows_t, rows_t)],
        )
        plsc.subcore_barrier()

        pltpu.async_copy(z_hbm.at[sidx.at[0]], gb0, sem0)

        dblks = (dblk_a, dblk_b)
        for b in range(nblk):
            dcur = dblks[b % 2]
            dnxt = dblks[(b + 1) % 2]
            if b + 1 < nblk:
                pltpu.async_copy(
                    dst_hbm.at[pl.ds(base + (b + 1) * BI, BI)], dnxt, semd
                )

            @pl.loop(0, BI, step=2)
            def _(k, b=b, dcur=dcur):
                g = b * BI + k
                pltpu.make_async_copy(z_hbm.at[sidx.at[g]], gb0, sem0).wait()
                pltpu.async_copy(z_hbm.at[sidx.at[g + 1]], gb1, sem1)
                pltpu.sync_copy(gb0, acc.at[dcur.at[k]], add=True)
                pltpu.make_async_copy(z_hbm.at[sidx.at[g + 1]], gb1, sem1).wait()

                @pl.when(g + 2 < erows_w)
                def _():
                    pltpu.async_copy(z_hbm.at[sidx.at[g + 2]], gb0, sem0)

                pltpu.sync_copy(gb1, acc.at[dcur.at[k + 1]], add=True)

            if b + 1 < nblk:
                pltpu.make_async_copy(
                    dst_hbm.at[pl.ds(base + (b + 1) * BI, BI)], dnxt, semd
                ).wait()

        plsc.subcore_barrier()
        pltpu.sync_copy(
            acc.at[pl.ds(sid * rows_t, rows_t)],
            out_hbm.at[cid, pl.ds(sid * rows_t, rows_t)],
        )

    return kscat


def _tc_scale_matmul(xp, w, dinv_col, block_rows=1024):
    n_pad, f_in = xp.shape
    f_out = w.shape[1]

    def body(x_ref, w_ref, d_ref, o_ref):
        o_ref[...] = d_ref[...] * jnp.dot(
            x_ref[...], w_ref[...], preferred_element_type=jnp.float32
        )

    return pl.pallas_call(
        body,
        grid=(n_pad // block_rows,),
        in_specs=[
            pl.BlockSpec((block_rows, f_in), lambda i: (i, 0)),
            pl.BlockSpec((f_in, f_out), lambda i: (0, 0)),
            pl.BlockSpec((block_rows, 1), lambda i: (i, 0)),
        ],
        out_specs=pl.BlockSpec((block_rows, f_out), lambda i: (i, 0)),
        out_shape=jax.ShapeDtypeStruct((n_pad, f_out), jnp.float32),
    )(xp, w, dinv_col)


def _tc_mid(s1, z1, dinv_col, b1_row, w2, block_rows=1024):
    _, n_pad, h = s1.shape
    c = w2.shape[1]

    def body(s_ref, z_ref, d_ref, b_ref, w_ref, o_ref):
        r = s_ref[0] + s_ref[1] + z_ref[...]
        act = jnp.maximum(d_ref[...] * r + b_ref[...], 0.0)
        res = d_ref[...] * jnp.dot(
            act, w_ref[...], preferred_element_type=jnp.float32
        )
        o_ref[...] = jnp.concatenate([res, jnp.zeros_like(res)], axis=-1)

    return pl.pallas_call(
        body,
        grid=(n_pad // block_rows,),
        in_specs=[
            pl.BlockSpec((2, block_rows, h), lambda i: (0, i, 0)),
            pl.BlockSpec((block_rows, h), lambda i: (i, 0)),
            pl.BlockSpec((block_rows, 1), lambda i: (i, 0)),
            pl.BlockSpec((1, h), lambda i: (0, 0)),
            pl.BlockSpec((h, c), lambda i: (0, 0)),
        ],
        out_specs=pl.BlockSpec((block_rows, 2 * c), lambda i: (i, 0)),
        out_shape=jax.ShapeDtypeStruct((n_pad, 2 * c), jnp.float32),
    )(s1, z1, dinv_col, b1_row, w2)


def _tc_final(s2, z2, dinv_col, b2_row, c, block_rows=1024):
    _, n_pad, cp = s2.shape

    def body(s_ref, z_ref, d_ref, b_ref, o_ref):
        r = (s_ref[0] + s_ref[1] + z_ref[...])[:, :c]
        o_ref[...] = d_ref[...] * r + b_ref[...]

    return pl.pallas_call(
        body,
        grid=(n_pad // block_rows,),
        in_specs=[
            pl.BlockSpec((2, block_rows, cp), lambda i: (0, i, 0)),
            pl.BlockSpec((block_rows, cp), lambda i: (i, 0)),
            pl.BlockSpec((block_rows, 1), lambda i: (i, 0)),
            pl.BlockSpec((1, c), lambda i: (0, 0)),
        ],
        out_specs=pl.BlockSpec((block_rows, c), lambda i: (i, 0)),
        out_shape=jax.ShapeDtypeStruct((n_pad, c), jnp.float32),
    )(s2, z2, dinv_col, b2_row)


def kernel(x, edge_index, W1, b1, W2, b2):
    n, f_in = x.shape
    e = edge_index.shape[1]
    h = W1.shape[1]
    c = W2.shape[1]

    n_pad = ((n + NC * NS * 8 - 1) // (NC * NS * 8)) * (NC * NS * 8)
    chunk = NC * NS * BI * LANE
    e_pad = ((e + chunk - 1) // chunk) * chunk

    src = edge_index[0]
    dst = edge_index[1]
    pad_e = e_pad - e
    srcp = jnp.concatenate(
        [src, jnp.full((pad_e,), n_pad - 1, jnp.int32)]
    ).reshape(e_pad // LANE, LANE)
    dstp = jnp.concatenate(
        [dst, jnp.full((pad_e,), n_pad - 2, jnp.int32)]
    ).reshape(e_pad // LANE, LANE)

    xp = jnp.zeros((n_pad, f_in), jnp.float32).at[:n].set(x)
    halves = jnp.full((n_pad,), 0.5, jnp.float32)
    ones_lane = jnp.ones((LANE,), jnp.float32)

    erows = e_pad // LANE
    deg_parts = _make_deg_kernel(n_pad, erows)(dstp, halves, ones_lane)
    dinv_col = lax.rsqrt(deg_parts[0] + deg_parts[1])[:, None]

    z1 = _tc_scale_matmul(xp, W1, dinv_col)
    s1 = _make_edge_scatter(n_pad, erows, h)(
        srcp, dstp, z1, jnp.zeros((n_pad, h), jnp.float32)
    )
    z2 = _tc_mid(s1, z1, dinv_col, b1.reshape(1, h), W2)
    s2 = _make_edge_scatter(n_pad, erows, 2 * c)(
        srcp, dstp, z2, jnp.zeros((n_pad, 2 * c), jnp.float32)
    )
    out = _tc_final(s2, z2, dinv_col, b2.reshape(1, c), c)
    return out[:n]


# speedup vs baseline: 10.9246x; 1.0964x over previous
import functools

import jax
import jax.numpy as jnp
from jax import lax
from jax.experimental import pallas as pl
from jax.experimental.pallas import tpu as pltpu
from jax.experimental.pallas import tpu_sc as plsc

NC = 2
NS = 16
LANE = 128
BI = 8


def _mesh():
    return plsc.VectorSubcoreMesh(
        core_axis_name="c", subcore_axis_name="s", num_cores=NC, num_subcores=NS
    )


def _make_deg_kernel(n_pad, erows):
    erows_w = erows // (NC * NS)
    rows_t = n_pad // NS

    @functools.partial(
        pl.kernel,
        out_type=jax.ShapeDtypeStruct((NC, n_pad), jnp.float32),
        mesh=_mesh(),
        scratch_types=[
            pltpu.VMEM((erows_w, LANE), jnp.int32),
            pltpu.VMEM((LANE,), jnp.float32),
            pltpu.VMEM_SHARED((n_pad,), jnp.float32),
        ],
    )
    def kdeg(dst_hbm, halves_hbm, ones_hbm, out_hbm, didx, onesv, acc):
        cid = lax.axis_index("c")
        sid = lax.axis_index("s")
        w = cid * NS + sid
        pltpu.sync_copy(dst_hbm.at[pl.ds(w * erows_w, erows_w)], didx)
        pltpu.sync_copy(ones_hbm, onesv)
        pltpu.sync_copy(
            halves_hbm.at[pl.ds(sid * rows_t, rows_t)],
            acc.at[pl.ds(sid * rows_t, rows_t)],
        )
        plsc.subcore_barrier()

        @pl.loop(0, erows_w)
        def _(i):
            pltpu.sync_copy(onesv, acc.at[didx.at[i]], add=True)

        plsc.subcore_barrier()
        pltpu.sync_copy(
            acc.at[pl.ds(sid * rows_t, rows_t)],
            out_hbm.at[cid, pl.ds(sid * rows_t, rows_t)],
        )

    return kdeg


def _make_edge_scatter(n_pad, erows, feat, tc_tiling=True):
    erows_w = erows // (NC * NS)
    rows_t = n_pad // NS
    nblk = erows_w // BI

    @functools.partial(
        pl.kernel,
        out_type=jax.ShapeDtypeStruct((NC, n_pad, feat), jnp.float32),
        mesh=_mesh(),
        compiler_params=pltpu.CompilerParams(use_tc_tiling_on_sc=tc_tiling),
        scratch_types=[
            pltpu.VMEM((erows_w, LANE), jnp.int32),
            pltpu.VMEM((BI, LANE), jnp.int32),
            pltpu.VMEM((BI, LANE), jnp.int32),
            pltpu.VMEM((LANE, feat), jnp.float32),
            pltpu.VMEM((LANE, feat), jnp.float32),
            pltpu.VMEM_SHARED((n_pad, feat), jnp.float32),
            pltpu.SemaphoreType.DMA,
            pltpu.SemaphoreType.DMA,
            pltpu.SemaphoreType.DMA,
        ],
    )
    def kscat(src_hbm, dst_hbm, z_hbm, zeros_hbm, out_hbm,
              sidx, dblk_a, dblk_b, gb0, gb1, acc, sem0, sem1, semd):
        cid = lax.axis_index("c")
        sid = lax.axis_index("s")
        w = cid * NS + sid
        base = w * erows_w
        pltpu.sync_copy(src_hbm.at[pl.ds(base, erows_w)], sidx)
        pltpu.sync_copy(dst_hbm.at[pl.ds(base, BI)], dblk_a)
        pltpu.sync_copy(
            zeros_hbm.at[pl.ds(sid * rows_t, rows_t)],
            acc.at[pl.ds(sid * rows_t, rows_t)],
        )
        plsc.subcore_barrier()

        pltpu.async_copy(z_hbm.at[sidx.at[0]], gb0, sem0)

        dblks = (dblk_a, dblk_b)
        for b in range(nblk):
            dcur = dblks[b % 2]
            dnxt = dblks[(b + 1) % 2]
            if b + 1 < nblk:
                pltpu.async_copy(
                    dst_hbm.at[pl.ds(base + (b + 1) * BI, BI)], dnxt, semd
                )

            @pl.loop(0, BI, step=2)
            def _(k, b=b, dcur=dcur):
                g = b * BI + k
                pltpu.make_async_copy(z_hbm.at[sidx.at[g]], gb0, sem0).wait()
                pltpu.async_copy(z_hbm.at[sidx.at[g + 1]], gb1, sem1)
                pltpu.sync_copy(gb0, acc.at[dcur.at[k]], add=True)
                pltpu.make_async_copy(z_hbm.at[sidx.at[g + 1]], gb1, sem1).wait()

                @pl.when(g + 2 < erows_w)
                def _():
                    pltpu.async_copy(z_hbm.at[sidx.at[g + 2]], gb0, sem0)

                pltpu.sync_copy(gb1, acc.at[dcur.at[k + 1]], add=True)

            if b + 1 < nblk:
                pltpu.make_async_copy(
                    dst_hbm.at[pl.ds(base + (b + 1) * BI, BI)], dnxt, semd
                ).wait()

        plsc.subcore_barrier()
        pltpu.sync_copy(
            acc.at[pl.ds(sid * rows_t, rows_t)],
            out_hbm.at[cid, pl.ds(sid * rows_t, rows_t)],
        )

    return kscat


def _tc_scale_matmul(xp, w, dinv_col, block_rows=1024):
    n_pad, f_in = xp.shape
    f_out = w.shape[1]

    def body(x_ref, w_ref, d_ref, o_ref):
        o_ref[...] = d_ref[...] * jnp.dot(
            x_ref[...], w_ref[...], preferred_element_type=jnp.float32
        )

    return pl.pallas_call(
        body,
        grid=(n_pad // block_rows,),
        in_specs=[
            pl.BlockSpec((block_rows, f_in), lambda i: (i, 0)),
            pl.BlockSpec((f_in, f_out), lambda i: (0, 0)),
            pl.BlockSpec((block_rows, 1), lambda i: (i, 0)),
        ],
        out_specs=pl.BlockSpec((block_rows, f_out), lambda i: (i, 0)),
        out_shape=jax.ShapeDtypeStruct((n_pad, f_out), jnp.float32),
    )(xp, w, dinv_col)


def _tc_mid(s1, z1, dinv_col, b1_row, w2, block_rows=1024):
    _, n_pad, h = s1.shape
    c = w2.shape[1]

    def body(s_ref, z_ref, d_ref, b_ref, w_ref, o_ref):
        r = s_ref[0] + s_ref[1] + z_ref[...]
        act = jnp.maximum(d_ref[...] * r + b_ref[...], 0.0)
        o_ref[...] = d_ref[...] * jnp.dot(
            act, w_ref[...], preferred_element_type=jnp.float32
        )

    return pl.pallas_call(
        body,
        grid=(n_pad // block_rows,),
        in_specs=[
            pl.BlockSpec((2, block_rows, h), lambda i: (0, i, 0)),
            pl.BlockSpec((block_rows, h), lambda i: (i, 0)),
            pl.BlockSpec((block_rows, 1), lambda i: (i, 0)),
            pl.BlockSpec((1, h), lambda i: (0, 0)),
            pl.BlockSpec((h, c), lambda i: (0, 0)),
        ],
        out_specs=pl.BlockSpec((block_rows, c), lambda i: (i, 0)),
        out_shape=jax.ShapeDtypeStruct((n_pad, c), jnp.float32),
    )(s1, z1, dinv_col, b1_row, w2)


def _tc_final(s2, z2, dinv_col, b2_row, c, block_rows=1024):
    _, n_pad, cp = s2.shape

    def body(s_ref, z_ref, d_ref, b_ref, o_ref):
        r = s_ref[0] + s_ref[1] + z_ref[...]
        o_ref[...] = d_ref[...] * r + b_ref[...]

    return pl.pallas_call(
        body,
        grid=(n_pad // block_rows,),
        in_specs=[
            pl.BlockSpec((2, block_rows, cp), lambda i: (0, i, 0)),
            pl.BlockSpec((block_rows, cp), lambda i: (i, 0)),
            pl.BlockSpec((block_rows, 1), lambda i: (i, 0)),
            pl.BlockSpec((1, c), lambda i: (0, 0)),
        ],
        out_specs=pl.BlockSpec((block_rows, c), lambda i: (i, 0)),
        out_shape=jax.ShapeDtypeStruct((n_pad, c), jnp.float32),
    )(s2, z2, dinv_col, b2_row)


def kernel(x, edge_index, W1, b1, W2, b2):
    n, f_in = x.shape
    e = edge_index.shape[1]
    h = W1.shape[1]
    c = W2.shape[1]

    n_pad = ((n + NC * NS * 8 - 1) // (NC * NS * 8)) * (NC * NS * 8)
    chunk = NC * NS * BI * LANE
    e_pad = ((e + chunk - 1) // chunk) * chunk

    src = edge_index[0]
    dst = edge_index[1]
    pad_e = e_pad - e
    srcp = jnp.concatenate(
        [src, jnp.full((pad_e,), n_pad - 1, jnp.int32)]
    ).reshape(e_pad // LANE, LANE)
    dstp = jnp.concatenate(
        [dst, jnp.full((pad_e,), n_pad - 2, jnp.int32)]
    ).reshape(e_pad // LANE, LANE)

    xp = jnp.zeros((n_pad, f_in), jnp.float32).at[:n].set(x)
    halves = jnp.full((n_pad,), 0.5, jnp.float32)
    ones_lane = jnp.ones((LANE,), jnp.float32)

    erows = e_pad // LANE
    deg_parts = _make_deg_kernel(n_pad, erows)(dstp, halves, ones_lane)
    dinv_col = lax.rsqrt(deg_parts[0] + deg_parts[1])[:, None]

    z1 = _tc_scale_matmul(xp, W1, dinv_col)
    s1 = _make_edge_scatter(n_pad, erows, h)(
        srcp, dstp, z1, jnp.zeros((n_pad, h), jnp.float32)
    )
    z2 = _tc_mid(s1, z1, dinv_col, b1.reshape(1, h), W2)
    s2 = _make_edge_scatter(n_pad, erows, c, tc_tiling=False)(
        srcp, dstp, z2, jnp.zeros((n_pad, c), jnp.float32)
    )
    out = _tc_final(s2, z2, dinv_col, b2.reshape(1, c), c)
    return out[:n]


# speedup vs baseline: 11.1864x; 1.0240x over previous
import functools

import jax
import jax.numpy as jnp
from jax import lax
from jax.experimental import pallas as pl
from jax.experimental.pallas import tpu as pltpu
from jax.experimental.pallas import tpu_sc as plsc

NC = 2
NS = 16
LANE = 128
BI = 8


def _mesh():
    return plsc.VectorSubcoreMesh(
        core_axis_name="c", subcore_axis_name="s", num_cores=NC, num_subcores=NS
    )


def _make_deg_kernel(n_pad, erows):
    erows_w = erows // (NC * NS)
    rows_t = n_pad // NS

    @functools.partial(
        pl.kernel,
        out_type=jax.ShapeDtypeStruct((NC, n_pad), jnp.float32),
        mesh=_mesh(),
        scratch_types=[
            pltpu.VMEM((erows_w, LANE), jnp.int32),
            pltpu.VMEM((LANE,), jnp.float32),
            pltpu.VMEM_SHARED((n_pad,), jnp.float32),
        ],
    )
    def kdeg(dst_hbm, halves_hbm, ones_hbm, out_hbm, didx, onesv, acc):
        cid = lax.axis_index("c")
        sid = lax.axis_index("s")
        w = cid * NS + sid
        pltpu.sync_copy(dst_hbm.at[pl.ds(w * erows_w, erows_w)], didx)
        pltpu.sync_copy(ones_hbm, onesv)
        pltpu.sync_copy(
            halves_hbm.at[pl.ds(sid * rows_t, rows_t)],
            acc.at[pl.ds(sid * rows_t, rows_t)],
        )
        plsc.subcore_barrier()

        @pl.loop(0, erows_w)
        def _(i):
            pltpu.sync_copy(onesv, acc.at[didx.at[i]], add=True)

        plsc.subcore_barrier()
        pltpu.sync_copy(
            acc.at[pl.ds(sid * rows_t, rows_t)],
            out_hbm.at[cid, pl.ds(sid * rows_t, rows_t)],
        )

    return kdeg


def _make_edge_scatter(n_pad, erows, feat, tc_tiling=True):
    erows_w = erows // (NC * NS)
    rows_t = n_pad // NS
    nblk = erows_w // BI

    @functools.partial(
        pl.kernel,
        out_type=jax.ShapeDtypeStruct((NC, n_pad, feat), jnp.float32),
        mesh=_mesh(),
        compiler_params=pltpu.CompilerParams(use_tc_tiling_on_sc=tc_tiling),
        scratch_types=[
            pltpu.VMEM((erows_w, LANE), jnp.int32),
            pltpu.VMEM((BI, LANE), jnp.int32),
            pltpu.VMEM((BI, LANE), jnp.int32),
            pltpu.VMEM((LANE, feat), jnp.float32),
            pltpu.VMEM((LANE, feat), jnp.float32),
            pltpu.VMEM_SHARED((n_pad, feat), jnp.float32),
            pltpu.SemaphoreType.DMA,
            pltpu.SemaphoreType.DMA,
            pltpu.SemaphoreType.DMA,
            pltpu.SemaphoreType.DMA,
            pltpu.SemaphoreType.DMA,
        ],
    )
    def kscat(src_hbm, dst_hbm, z_hbm, zeros_hbm, out_hbm,
              sidx, dblk_a, dblk_b, gb0, gb1, acc, sem0, sem1, semd,
              sems0, sems1):
        cid = lax.axis_index("c")
        sid = lax.axis_index("s")
        w = cid * NS + sid
        base = w * erows_w
        pltpu.sync_copy(src_hbm.at[pl.ds(base, erows_w)], sidx)
        pltpu.sync_copy(dst_hbm.at[pl.ds(base, BI)], dblk_a)
        pltpu.sync_copy(
            zeros_hbm.at[pl.ds(sid * rows_t, rows_t)],
            acc.at[pl.ds(sid * rows_t, rows_t)],
        )
        plsc.subcore_barrier()

        pltpu.async_copy(z_hbm.at[sidx.at[0]], gb0, sem0)
        pltpu.async_copy(z_hbm.at[sidx.at[1]], gb1, sem1)

        dblks = (dblk_a, dblk_b)
        for b in range(nblk):
            dcur = dblks[b % 2]
            dnxt = dblks[(b + 1) % 2]
            if b + 1 < nblk:
                pltpu.async_copy(
                    dst_hbm.at[pl.ds(base + (b + 1) * BI, BI)], dnxt, semd
                )

            @pl.loop(0, BI, step=2)
            def _(k, b=b, dcur=dcur):
                g = b * BI + k
                pltpu.make_async_copy(z_hbm.at[sidx.at[g]], gb0, sem0).wait()
                sc0 = pltpu.async_copy(gb0, acc.at[dcur.at[k]], sems0, add=True)
                pltpu.make_async_copy(z_hbm.at[sidx.at[g + 1]], gb1, sem1).wait()
                sc1 = pltpu.async_copy(
                    gb1, acc.at[dcur.at[k + 1]], sems1, add=True
                )
                sc0.wait()

                @pl.when(g + 2 < erows_w)
                def _():
                    pltpu.async_copy(z_hbm.at[sidx.at[g + 2]], gb0, sem0)

                sc1.wait()

                @pl.when(g + 3 < erows_w)
                def _():
                    pltpu.async_copy(z_hbm.at[sidx.at[g + 3]], gb1, sem1)

            if b + 1 < nblk:
                pltpu.make_async_copy(
                    dst_hbm.at[pl.ds(base + (b + 1) * BI, BI)], dnxt, semd
                ).wait()

        plsc.subcore_barrier()
        pltpu.sync_copy(
            acc.at[pl.ds(sid * rows_t, rows_t)],
            out_hbm.at[cid, pl.ds(sid * rows_t, rows_t)],
        )

    return kscat


def _tc_scale_matmul(xp, w, dinv_col, block_rows=1024):
    n_pad, f_in = xp.shape
    f_out = w.shape[1]

    def body(x_ref, w_ref, d_ref, o_ref):
        o_ref[...] = d_ref[...] * jnp.dot(
            x_ref[...], w_ref[...], preferred_element_type=jnp.float32
        )

    return pl.pallas_call(
        body,
        grid=(n_pad // block_rows,),
        in_specs=[
            pl.BlockSpec((block_rows, f_in), lambda i: (i, 0)),
            pl.BlockSpec((f_in, f_out), lambda i: (0, 0)),
            pl.BlockSpec((block_rows, 1), lambda i: (i, 0)),
        ],
        out_specs=pl.BlockSpec((block_rows, f_out), lambda i: (i, 0)),
        out_shape=jax.ShapeDtypeStruct((n_pad, f_out), jnp.float32),
    )(xp, w, dinv_col)


def _tc_mid(s1, z1, dinv_col, b1_row, w2, block_rows=1024):
    _, n_pad, h = s1.shape
    c = w2.shape[1]

    def body(s_ref, z_ref, d_ref, b_ref, w_ref, o_ref):
        r = s_ref[0] + s_ref[1] + z_ref[...]
        act = jnp.maximum(d_ref[...] * r + b_ref[...], 0.0)
        o_ref[...] = d_ref[...] * jnp.dot(
            act, w_ref[...], preferred_element_type=jnp.float32
        )

    return pl.pallas_call(
        body,
        grid=(n_pad // block_rows,),
        in_specs=[
            pl.BlockSpec((2, block_rows, h), lambda i: (0, i, 0)),
            pl.BlockSpec((block_rows, h), lambda i: (i, 0)),
            pl.BlockSpec((block_rows, 1), lambda i: (i, 0)),
            pl.BlockSpec((1, h), lambda i: (0, 0)),
            pl.BlockSpec((h, c), lambda i: (0, 0)),
        ],
        out_specs=pl.BlockSpec((block_rows, c), lambda i: (i, 0)),
        out_shape=jax.ShapeDtypeStruct((n_pad, c), jnp.float32),
    )(s1, z1, dinv_col, b1_row, w2)


def _tc_final(s2, z2, dinv_col, b2_row, c, block_rows=1024):
    _, n_pad, cp = s2.shape

    def body(s_ref, z_ref, d_ref, b_ref, o_ref):
        r = s_ref[0] + s_ref[1] + z_ref[...]
        o_ref[...] = d_ref[...] * r + b_ref[...]

    return pl.pallas_call(
        body,
        grid=(n_pad // block_rows,),
        in_specs=[
            pl.BlockSpec((2, block_rows, cp), lambda i: (0, i, 0)),
            pl.BlockSpec((block_rows, cp), lambda i: (i, 0)),
            pl.BlockSpec((block_rows, 1), lambda i: (i, 0)),
            pl.BlockSpec((1, c), lambda i: (0, 0)),
        ],
        out_specs=pl.BlockSpec((block_rows, c), lambda i: (i, 0)),
        out_shape=jax.ShapeDtypeStruct((n_pad, c), jnp.float32),
    )(s2, z2, dinv_col, b2_row)


def kernel(x, edge_index, W1, b1, W2, b2):
    n, f_in = x.shape
    e = edge_index.shape[1]
    h = W1.shape[1]
    c = W2.shape[1]

    n_pad = ((n + NC * NS * 8 - 1) // (NC * NS * 8)) * (NC * NS * 8)
    chunk = NC * NS * BI * LANE
    e_pad = ((e + chunk - 1) // chunk) * chunk

    src = edge_index[0]
    dst = edge_index[1]
    pad_e = e_pad - e
    srcp = jnp.concatenate(
        [src, jnp.full((pad_e,), n_pad - 1, jnp.int32)]
    ).reshape(e_pad // LANE, LANE)
    dstp = jnp.concatenate(
        [dst, jnp.full((pad_e,), n_pad - 2, jnp.int32)]
    ).reshape(e_pad // LANE, LANE)

    xp = jnp.zeros((n_pad, f_in), jnp.float32).at[:n].set(x)
    halves = jnp.full((n_pad,), 0.5, jnp.float32)
    ones_lane = jnp.ones((LANE,), jnp.float32)

    erows = e_pad // LANE
    deg_parts = _make_deg_kernel(n_pad, erows)(dstp, halves, ones_lane)
    dinv_col = lax.rsqrt(deg_parts[0] + deg_parts[1])[:, None]

    z1 = _tc_scale_matmul(xp, W1, dinv_col)
    s1 = _make_edge_scatter(n_pad, erows, h)(
        srcp, dstp, z1, jnp.zeros((n_pad, h), jnp.float32)
    )
    z2 = _tc_mid(s1, z1, dinv_col, b1.reshape(1, h), W2)
    s2 = _make_edge_scatter(n_pad, erows, c, tc_tiling=False)(
        srcp, dstp, z2, jnp.zeros((n_pad, c), jnp.float32)
    )
    out = _tc_final(s2, z2, dinv_col, b2.reshape(1, c), c)
    return out[:n]


# speedup vs baseline: 28.6700x; 2.5629x over previous
import functools

import jax
import jax.numpy as jnp
from jax import lax
from jax.experimental import pallas as pl
from jax.experimental.pallas import tpu as pltpu
from jax.experimental.pallas import tpu_sc as plsc

NC = 2
NS = 16
LANE = 128
BI = 8


def _mesh():
    return plsc.VectorSubcoreMesh(
        core_axis_name="c", subcore_axis_name="s", num_cores=NC, num_subcores=NS
    )


def _make_deg_kernel(n_pad, erows):
    erows_w = erows // (NC * NS)
    rows_t = n_pad // NS

    @functools.partial(
        pl.kernel,
        out_type=jax.ShapeDtypeStruct((NC, n_pad), jnp.float32),
        mesh=_mesh(),
        scratch_types=[
            pltpu.VMEM((erows_w, LANE), jnp.int32),
            pltpu.VMEM((LANE,), jnp.float32),
            pltpu.VMEM_SHARED((n_pad,), jnp.float32),
        ],
    )
    def kdeg(dst_hbm, halves_hbm, ones_hbm, out_hbm, didx, onesv, acc):
        cid = lax.axis_index("c")
        sid = lax.axis_index("s")
        w = cid * NS + sid
        pltpu.sync_copy(dst_hbm.at[pl.ds(w * erows_w, erows_w)], didx)
        pltpu.sync_copy(ones_hbm, onesv)
        pltpu.sync_copy(
            halves_hbm.at[pl.ds(sid * rows_t, rows_t)],
            acc.at[pl.ds(sid * rows_t, rows_t)],
        )
        plsc.subcore_barrier()

        @pl.loop(0, erows_w)
        def _(i):
            pltpu.sync_copy(onesv, acc.at[didx.at[i]], add=True)

        plsc.subcore_barrier()
        pltpu.sync_copy(
            acc.at[pl.ds(sid * rows_t, rows_t)],
            out_hbm.at[cid, pl.ds(sid * rows_t, rows_t)],
        )

    return kdeg


def _make_edge_scatter(n_pad, erows, feat, tc_tiling=True):
    erows_w = erows // (NC * NS)
    rows_t = n_pad // NS
    nblk = erows_w // BI

    @functools.partial(
        pl.kernel,
        out_type=jax.ShapeDtypeStruct((NC, n_pad, feat), jnp.float32),
        mesh=_mesh(),
        compiler_params=pltpu.CompilerParams(use_tc_tiling_on_sc=tc_tiling),
        scratch_types=[
            pltpu.VMEM((erows_w, LANE), jnp.int32),
            pltpu.VMEM((BI, LANE), jnp.int32),
            pltpu.VMEM((BI, LANE), jnp.int32),
            pltpu.VMEM((LANE, feat), jnp.float32),
            pltpu.VMEM((LANE, feat), jnp.float32),
            pltpu.VMEM_SHARED((n_pad, feat), jnp.float32),
            pltpu.SemaphoreType.DMA,
            pltpu.SemaphoreType.DMA,
            pltpu.SemaphoreType.DMA,
            pltpu.SemaphoreType.DMA,
            pltpu.SemaphoreType.DMA,
        ],
    )
    def kscat(src_hbm, dst_hbm, z_hbm, zeros_hbm, out_hbm,
              sidx, dblk_a, dblk_b, gb0, gb1, acc, sem0, sem1, semd,
              sems0, sems1):
        cid = lax.axis_index("c")
        sid = lax.axis_index("s")
        w = cid * NS + sid
        base = w * erows_w
        pltpu.sync_copy(src_hbm.at[pl.ds(base, erows_w)], sidx)
        pltpu.sync_copy(dst_hbm.at[pl.ds(base, BI)], dblk_a)
        pltpu.sync_copy(
            zeros_hbm.at[pl.ds(sid * rows_t, rows_t)],
            acc.at[pl.ds(sid * rows_t, rows_t)],
        )
        plsc.subcore_barrier()

        pltpu.async_copy(z_hbm.at[sidx.at[0]], gb0, sem0)
        pltpu.async_copy(z_hbm.at[sidx.at[1]], gb1, sem1)

        dblks = (dblk_a, dblk_b)
        for b in range(nblk):
            dcur = dblks[b % 2]
            dnxt = dblks[(b + 1) % 2]
            if b + 1 < nblk:
                pltpu.async_copy(
                    dst_hbm.at[pl.ds(base + (b + 1) * BI, BI)], dnxt, semd
                )

            @pl.loop(0, BI, step=2)
            def _(k, b=b, dcur=dcur):
                g = b * BI + k
                pltpu.make_async_copy(z_hbm.at[sidx.at[g]], gb0, sem0).wait()
                sc0 = pltpu.async_copy(gb0, acc.at[dcur.at[k]], sems0, add=True)
                pltpu.make_async_copy(z_hbm.at[sidx.at[g + 1]], gb1, sem1).wait()
                sc1 = pltpu.async_copy(
                    gb1, acc.at[dcur.at[k + 1]], sems1, add=True
                )
                sc0.wait()

                @pl.when(g + 2 < erows_w)
                def _():
                    pltpu.async_copy(z_hbm.at[sidx.at[g + 2]], gb0, sem0)

                sc1.wait()

                @pl.when(g + 3 < erows_w)
                def _():
                    pltpu.async_copy(z_hbm.at[sidx.at[g + 3]], gb1, sem1)

            if b + 1 < nblk:
                pltpu.make_async_copy(
                    dst_hbm.at[pl.ds(base + (b + 1) * BI, BI)], dnxt, semd
                ).wait()

        plsc.subcore_barrier()
        pltpu.sync_copy(
            acc.at[pl.ds(sid * rows_t, rows_t)],
            out_hbm.at[cid, pl.ds(sid * rows_t, rows_t)],
        )

    return kscat


def _tc_scale_matmul(xp, w, dinv_col, block_rows=1024):
    n_pad, f_in = xp.shape
    f_out = w.shape[1]

    def body(x_ref, w_ref, d_ref, o_ref):
        o_ref[...] = d_ref[...] * jnp.dot(
            x_ref[...], w_ref[...], preferred_element_type=jnp.float32
        )

    return pl.pallas_call(
        body,
        grid=(n_pad // block_rows,),
        in_specs=[
            pl.BlockSpec((block_rows, f_in), lambda i: (i, 0)),
            pl.BlockSpec((f_in, f_out), lambda i: (0, 0)),
            pl.BlockSpec((block_rows, 1), lambda i: (i, 0)),
        ],
        out_specs=pl.BlockSpec((block_rows, f_out), lambda i: (i, 0)),
        out_shape=jax.ShapeDtypeStruct((n_pad, f_out), jnp.float32),
    )(xp, w, dinv_col)


def _tc_mid(s1, z1, dinv_col, b1_row, w2, block_rows=1024):
    _, n_pad, h = s1.shape
    c = w2.shape[1]

    def body(s_ref, z_ref, d_ref, b_ref, w_ref, o_ref):
        r = s_ref[0] + s_ref[1] + z_ref[...]
        act = jnp.maximum(d_ref[...] * r + b_ref[...], 0.0)
        o_ref[...] = d_ref[...] * jnp.dot(
            act, w_ref[...], preferred_element_type=jnp.float32
        )

    return pl.pallas_call(
        body,
        grid=(n_pad // block_rows,),
        in_specs=[
            pl.BlockSpec((2, block_rows, h), lambda i: (0, i, 0)),
            pl.BlockSpec((block_rows, h), lambda i: (i, 0)),
            pl.BlockSpec((block_rows, 1), lambda i: (i, 0)),
            pl.BlockSpec((1, h), lambda i: (0, 0)),
            pl.BlockSpec((h, c), lambda i: (0, 0)),
        ],
        out_specs=pl.BlockSpec((block_rows, c), lambda i: (i, 0)),
        out_shape=jax.ShapeDtypeStruct((n_pad, c), jnp.float32),
    )(s1, z1, dinv_col, b1_row, w2)


def _tc_final(s2, z2, dinv_col, b2_row, c, block_rows=1024):
    _, n_pad, cp = s2.shape

    def body(s_ref, z_ref, d_ref, b_ref, o_ref):
        r = s_ref[0] + s_ref[1] + z_ref[...]
        o_ref[...] = d_ref[...] * r + b_ref[...]

    return pl.pallas_call(
        body,
        grid=(n_pad // block_rows,),
        in_specs=[
            pl.BlockSpec((2, block_rows, cp), lambda i: (0, i, 0)),
            pl.BlockSpec((block_rows, cp), lambda i: (i, 0)),
            pl.BlockSpec((block_rows, 1), lambda i: (i, 0)),
            pl.BlockSpec((1, c), lambda i: (0, 0)),
        ],
        out_specs=pl.BlockSpec((block_rows, c), lambda i: (i, 0)),
        out_shape=jax.ShapeDtypeStruct((n_pad, c), jnp.float32),
    )(s2, z2, dinv_col, b2_row)


def kernel(x, edge_index, W1, b1, W2, b2):
    n, f_in = x.shape
    e = edge_index.shape[1]
    h = W1.shape[1]
    c = W2.shape[1]

    n_pad = ((n + NC * NS * 8 - 1) // (NC * NS * 8)) * (NC * NS * 8)
    chunk = NC * NS * BI * LANE
    e_pad = ((e + chunk - 1) // chunk) * chunk

    src = edge_index[0]
    dst = edge_index[1]
    pad_e = e_pad - e
    pad_idx = (n + (jnp.arange(pad_e, dtype=jnp.int32) % (n_pad - n))).astype(
        jnp.int32
    )
    srcp = jnp.concatenate([src, pad_idx]).reshape(e_pad // LANE, LANE)
    dstp = jnp.concatenate([dst, pad_idx]).reshape(e_pad // LANE, LANE)

    xp = jnp.zeros((n_pad, f_in), jnp.float32).at[:n].set(x)
    halves = jnp.full((n_pad,), 0.5, jnp.float32)
    ones_lane = jnp.ones((LANE,), jnp.float32)

    erows = e_pad // LANE
    deg_parts = _make_deg_kernel(n_pad, erows)(dstp, halves, ones_lane)
    dinv_col = lax.rsqrt(deg_parts[0] + deg_parts[1])[:, None]

    z1 = _tc_scale_matmul(xp, W1, dinv_col)
    s1 = _make_edge_scatter(n_pad, erows, h)(
        srcp, dstp, z1, jnp.zeros((n_pad, h), jnp.float32)
    )
    z2 = _tc_mid(s1, z1, dinv_col, b1.reshape(1, h), W2)
    s2 = _make_edge_scatter(n_pad, erows, c, tc_tiling=False)(
        srcp, dstp, z2, jnp.zeros((n_pad, c), jnp.float32)
    )
    out = _tc_final(s2, z2, dinv_col, b2.reshape(1, c), c)
    return out[:n]


# speedup vs baseline: 30.8389x; 1.0756x over previous
import functools

import jax
import jax.numpy as jnp
from jax import lax
from jax.experimental import pallas as pl
from jax.experimental.pallas import tpu as pltpu
from jax.experimental.pallas import tpu_sc as plsc

NC = 2
NS = 16
LANE = 128
BI = 8


def _mesh():
    return plsc.VectorSubcoreMesh(
        core_axis_name="c", subcore_axis_name="s", num_cores=NC, num_subcores=NS
    )


def _make_deg_kernel(n_pad, erows):
    erows_w = erows // (NC * NS)
    rows_t = n_pad // NS

    @functools.partial(
        pl.kernel,
        out_type=jax.ShapeDtypeStruct((NC, n_pad), jnp.float32),
        mesh=_mesh(),
        scratch_types=[
            pltpu.VMEM((erows_w, LANE), jnp.int32),
            pltpu.VMEM((LANE,), jnp.float32),
            pltpu.VMEM_SHARED((n_pad,), jnp.float32),
        ],
    )
    def kdeg(dst_hbm, halves_hbm, ones_hbm, out_hbm, didx, onesv, acc):
        cid = lax.axis_index("c")
        sid = lax.axis_index("s")
        w = cid * NS + sid
        pltpu.sync_copy(dst_hbm.at[pl.ds(w * erows_w, erows_w)], didx)
        pltpu.sync_copy(ones_hbm, onesv)
        pltpu.sync_copy(
            halves_hbm.at[pl.ds(sid * rows_t, rows_t)],
            acc.at[pl.ds(sid * rows_t, rows_t)],
        )
        plsc.subcore_barrier()

        @pl.loop(0, erows_w)
        def _(i):
            pltpu.sync_copy(onesv, acc.at[didx.at[i]], add=True)

        plsc.subcore_barrier()
        pltpu.sync_copy(
            acc.at[pl.ds(sid * rows_t, rows_t)],
            out_hbm.at[cid, pl.ds(sid * rows_t, rows_t)],
        )

    return kdeg


def _make_edge_scatter(n_pad, erows, feat, tc_tiling=True, ring=2, bi=BI):
    erows_w = erows // (NC * NS)
    rows_t = n_pad // NS
    nblk = erows_w // bi
    assert bi % ring == 0 and erows_w % bi == 0

    @functools.partial(
        pl.kernel,
        out_type=jax.ShapeDtypeStruct((NC, n_pad, feat), jnp.float32),
        mesh=_mesh(),
        compiler_params=pltpu.CompilerParams(use_tc_tiling_on_sc=tc_tiling),
        scratch_types=[
            pltpu.VMEM((erows_w, LANE), jnp.int32),
            pltpu.VMEM((bi, LANE), jnp.int32),
            pltpu.VMEM((bi, LANE), jnp.int32),
            [pltpu.VMEM((LANE, feat), jnp.float32) for _ in range(ring)],
            pltpu.VMEM_SHARED((n_pad, feat), jnp.float32),
            [pltpu.SemaphoreType.DMA for _ in range(ring)],
            [pltpu.SemaphoreType.DMA for _ in range(ring)],
            pltpu.SemaphoreType.DMA,
        ],
    )
    def kscat(src_hbm, dst_hbm, z_hbm, zeros_hbm, out_hbm,
              sidx, dblk_a, dblk_b, gbs, acc, semg, sems, semd):
        cid = lax.axis_index("c")
        sid = lax.axis_index("s")
        w = cid * NS + sid
        base = w * erows_w
        pltpu.sync_copy(src_hbm.at[pl.ds(base, erows_w)], sidx)
        pltpu.sync_copy(dst_hbm.at[pl.ds(base, bi)], dblk_a)
        pltpu.sync_copy(
            zeros_hbm.at[pl.ds(sid * rows_t, rows_t)],
            acc.at[pl.ds(sid * rows_t, rows_t)],
        )
        plsc.subcore_barrier()

        for r in range(ring):
            pltpu.async_copy(z_hbm.at[sidx.at[r]], gbs[r], semg[r])

        dblks = (dblk_a, dblk_b)
        for b in range(nblk):
            dcur = dblks[b % 2]
            dnxt = dblks[(b + 1) % 2]
            if b + 1 < nblk:
                pltpu.async_copy(
                    dst_hbm.at[pl.ds(base + (b + 1) * bi, bi)], dnxt, semd
                )

            @pl.loop(0, bi, step=ring)
            def _(k, b=b, dcur=dcur):
                scd = []
                for r in range(ring):
                    g = b * bi + k + r
                    pltpu.make_async_copy(
                        z_hbm.at[sidx.at[g]], gbs[r], semg[r]
                    ).wait()
                    scd.append(pltpu.async_copy(
                        gbs[r], acc.at[dcur.at[k + r]], sems[r], add=True
                    ))
                for r in range(ring):
                    g = b * bi + k + r
                    scd[r].wait()

                    @pl.when(g + ring < erows_w)
                    def _(g=g, r=r):
                        pltpu.async_copy(
                            z_hbm.at[sidx.at[g + ring]], gbs[r], semg[r]
                        )

            if b + 1 < nblk:
                pltpu.make_async_copy(
                    dst_hbm.at[pl.ds(base + (b + 1) * bi, bi)], dnxt, semd
                ).wait()

        plsc.subcore_barrier()
        pltpu.sync_copy(
            acc.at[pl.ds(sid * rows_t, rows_t)],
            out_hbm.at[cid, pl.ds(sid * rows_t, rows_t)],
        )

    return kscat


def _tc_matmul(xp, w, block_rows=1024):
    n_pad, f_in = xp.shape
    f_out = w.shape[1]

    def body(x_ref, w_ref, o_ref):
        o_ref[...] = jnp.dot(
            x_ref[...], w_ref[...], preferred_element_type=jnp.float32
        )

    return pl.pallas_call(
        body,
        grid=(n_pad // block_rows,),
        in_specs=[
            pl.BlockSpec((block_rows, f_in), lambda i: (i, 0)),
            pl.BlockSpec((f_in, f_out), lambda i: (0, 0)),
        ],
        out_specs=pl.BlockSpec((block_rows, f_out), lambda i: (i, 0)),
        out_shape=jax.ShapeDtypeStruct((n_pad, f_out), jnp.float32),
    )(xp, w)


def _tc_scale(y, dinv_col, block_rows=1024):
    n_pad, f_out = y.shape

    def body(y_ref, d_ref, o_ref):
        o_ref[...] = d_ref[...] * y_ref[...]

    return pl.pallas_call(
        body,
        grid=(n_pad // block_rows,),
        in_specs=[
            pl.BlockSpec((block_rows, f_out), lambda i: (i, 0)),
            pl.BlockSpec((block_rows, 1), lambda i: (i, 0)),
        ],
        out_specs=pl.BlockSpec((block_rows, f_out), lambda i: (i, 0)),
        out_shape=jax.ShapeDtypeStruct((n_pad, f_out), jnp.float32),
    )(y, dinv_col)


def _tc_mid(s1, z1, dinv_col, b1_row, w2, block_rows=1024):
    _, n_pad, h = s1.shape
    c = w2.shape[1]

    def body(s_ref, z_ref, d_ref, b_ref, w_ref, o_ref):
        r = s_ref[0] + s_ref[1] + z_ref[...]
        act = jnp.maximum(d_ref[...] * r + b_ref[...], 0.0)
        o_ref[...] = d_ref[...] * jnp.dot(
            act, w_ref[...], preferred_element_type=jnp.float32
        )

    return pl.pallas_call(
        body,
        grid=(n_pad // block_rows,),
        in_specs=[
            pl.BlockSpec((2, block_rows, h), lambda i: (0, i, 0)),
            pl.BlockSpec((block_rows, h), lambda i: (i, 0)),
            pl.BlockSpec((block_rows, 1), lambda i: (i, 0)),
            pl.BlockSpec((1, h), lambda i: (0, 0)),
            pl.BlockSpec((h, c), lambda i: (0, 0)),
        ],
        out_specs=pl.BlockSpec((block_rows, c), lambda i: (i, 0)),
        out_shape=jax.ShapeDtypeStruct((n_pad, c), jnp.float32),
    )(s1, z1, dinv_col, b1_row, w2)


def _tc_final(s2, z2, dinv_col, b2_row, c, block_rows=1024):
    _, n_pad, cp = s2.shape

    def body(s_ref, z_ref, d_ref, b_ref, o_ref):
        r = s_ref[0] + s_ref[1] + z_ref[...]
        o_ref[...] = d_ref[...] * r + b_ref[...]

    return pl.pallas_call(
        body,
        grid=(n_pad // block_rows,),
        in_specs=[
            pl.BlockSpec((2, block_rows, cp), lambda i: (0, i, 0)),
            pl.BlockSpec((block_rows, cp), lambda i: (i, 0)),
            pl.BlockSpec((block_rows, 1), lambda i: (i, 0)),
            pl.BlockSpec((1, c), lambda i: (0, 0)),
        ],
        out_specs=pl.BlockSpec((block_rows, c), lambda i: (i, 0)),
        out_shape=jax.ShapeDtypeStruct((n_pad, c), jnp.float32),
    )(s2, z2, dinv_col, b2_row)


def kernel(x, edge_index, W1, b1, W2, b2):
    n, f_in = x.shape
    e = edge_index.shape[1]
    h = W1.shape[1]
    c = W2.shape[1]

    n_pad = ((n + NC * NS * 8 - 1) // (NC * NS * 8)) * (NC * NS * 8)
    chunk = NC * NS * BI * LANE
    e_pad = ((e + chunk - 1) // chunk) * chunk

    src = edge_index[0]
    dst = edge_index[1]
    pad_e = e_pad - e
    pad_idx = (n + (jnp.arange(pad_e, dtype=jnp.int32) % (n_pad - n))).astype(
        jnp.int32
    )
    srcp = jnp.concatenate([src, pad_idx]).reshape(e_pad // LANE, LANE)
    dstp = jnp.concatenate([dst, pad_idx]).reshape(e_pad // LANE, LANE)

    xp = jnp.zeros((n_pad, f_in), jnp.float32).at[:n].set(x)
    halves = jnp.full((n_pad,), 0.5, jnp.float32)
    ones_lane = jnp.ones((LANE,), jnp.float32)

    erows = e_pad // LANE
    y1 = _tc_matmul(xp, W1)
    deg_parts = _make_deg_kernel(n_pad, erows)(dstp, halves, ones_lane)
    dinv_col = lax.rsqrt(deg_parts[0] + deg_parts[1])[:, None]

    z1 = _tc_scale(y1, dinv_col)
    s1 = _make_edge_scatter(n_pad, erows, h, ring=2, bi=16)(
        srcp, dstp, z1, jnp.zeros((n_pad, h), jnp.float32)
    )
    z2 = _tc_mid(s1, z1, dinv_col, b1.reshape(1, h), W2)
    s2 = _make_edge_scatter(n_pad, erows, c, tc_tiling=False, ring=4, bi=8)(
        srcp, dstp, z2, jnp.zeros((n_pad, c), jnp.float32)
    )
    out = _tc_final(s2, z2, dinv_col, b2.reshape(1, c), c)
    return out[:n]
